# baseline (speedup 1.0000x reference)
import jax
import jax.numpy as jnp
from jax.experimental import pallas as pl
from jax.experimental.pallas import tpu as pltpu

_TN = 512


def _body(x_ref, w_ref, b_ref, o_ref):
    k_tot, _, _ = x_ref.shape
    out_ch = w_ref.shape[1]
    tn = x_ref.shape[1]
    acc = jnp.zeros((tn, out_ch), jnp.float32)
    for k in range(k_tot):
        acc = acc + jax.lax.dot_general(
            x_ref[k], w_ref[k],
            (((1,), (1,)), ((), ())),
            preferred_element_type=jnp.float32)
    o_ref[...] = acc + jnp.sum(b_ref[...], axis=0)[None, :]


def kernel(inputs, weights, bias):
    k_tot, n, in_ch = inputs.shape
    out_ch = weights.shape[1]
    tn = min(_TN, n)
    return pl.pallas_call(
        _body,
        grid=(n // tn,),
        in_specs=[
            pl.BlockSpec((k_tot, tn, in_ch), lambda i: (0, i, 0)),
            pl.BlockSpec((k_tot, out_ch, in_ch), lambda i: (0, 0, 0)),
            pl.BlockSpec((k_tot, out_ch), lambda i: (0, 0)),
        ],
        out_specs=pl.BlockSpec((tn, out_ch), lambda i: (i, 0)),
        out_shape=jax.ShapeDtypeStruct((n, out_ch), jnp.float32),
        compiler_params=pltpu.CompilerParams(
            dimension_semantics=("parallel",),
        ),
    )(inputs, weights, bias)


# speedup vs baseline: 1.1685x; 1.1685x over previous
import jax
import jax.numpy as jnp
from jax.experimental import pallas as pl
from jax.experimental.pallas import tpu as pltpu

_TN = 1024


def _body(x_ref, w_ref, b_ref, o_ref):
    k_tot, _, _ = x_ref.shape
    out_ch = w_ref.shape[1]
    tn = x_ref.shape[1]
    acc = jnp.zeros((tn, out_ch), jnp.float32)
    for k in range(k_tot):
        acc = acc + jax.lax.dot_general(
            x_ref[k], w_ref[k],
            (((1,), (1,)), ((), ())),
            preferred_element_type=jnp.float32)
    o_ref[...] = acc + jnp.sum(b_ref[...], axis=0)[None, :]


def kernel(inputs, weights, bias):
    k_tot, n, in_ch = inputs.shape
    out_ch = weights.shape[1]
    tn = min(_TN, n)
    return pl.pallas_call(
        _body,
        grid=(n // tn,),
        in_specs=[
            pl.BlockSpec((k_tot, tn, in_ch), lambda i: (0, i, 0)),
            pl.BlockSpec((k_tot, out_ch, in_ch), lambda i: (0, 0, 0)),
            pl.BlockSpec((k_tot, out_ch), lambda i: (0, 0)),
        ],
        out_specs=pl.BlockSpec((tn, out_ch), lambda i: (i, 0)),
        out_shape=jax.ShapeDtypeStruct((n, out_ch), jnp.float32),
        compiler_params=pltpu.CompilerParams(
            dimension_semantics=("parallel",),
        ),
    )(inputs, weights, bias)
